# baseline (speedup 1.0000x reference)
import dataclasses
import functools

import jax
import jax.numpy as jnp
from jax.experimental import pallas as pl
from jax.experimental.pallas import tpu as pltpu
from jax.experimental.pallas import tpu_sc as plsc

_B, _S = 16384, 200
_LANES = 16
_RB = 8
_DC = 1024
_GRID_R = _S // _RB
_GRID_C = _B // _DC
_GRID = _GRID_R * _GRID_C


def _sc_lookup(xt, tbl_pad):
    mesh = plsc.VectorSubcoreMesh(core_axis_name="c", subcore_axis_name="s")
    cp = pltpu.CompilerParams()
    if "needs_layout_passes" in pltpu.CompilerParams.__dataclass_fields__:
        cp = dataclasses.replace(cp, needs_layout_passes=False)

    @functools.partial(
        pl.kernel,
        out_type=jax.ShapeDtypeStruct((_S, _B // 32, 128), jnp.float32),
        mesh=mesh,
        scratch_types=[pltpu.VMEM((576,), jnp.float32)],
        compiler_params=cp,
    )
    def k(x_hbm, tbl_hbm, out_hbm, ts_v):
        pltpu.sync_copy(tbl_hbm, ts_v)

        def body(x_vmem, o_vmem):
            lane = jax.lax.broadcasted_iota(jnp.int32, (_LANES,), 0)
            lane_c = [lane + _LANES * c for c in range(4)]

            @plsc.parallel_loop(0, _RB, unroll=2)
            def _(r):
                @plsc.parallel_loop(0, _DC // 128, unroll=4)
                def _(h):
                    for i in range(128 // _LANES):
                        xv = x_vmem[r, pl.ds(h * 128 + i * _LANES, _LANES)]
                        base = xv * 64
                        for c in range(4):
                            vals = plsc.load_gather(ts_v, [base + lane_c[c]])
                            o_vmem[r, h * 4 + c, pl.ds(i * _LANES, _LANES)] = vals

        pltpu.emit_pipeline(
            body,
            grid=(_GRID,),
            in_specs=[
                pl.BlockSpec(
                    (_RB, _DC),
                    lambda i: (i // _GRID_C, i % _GRID_C),
                    pipeline_mode=pl.Buffered(buffer_count=3),
                )
            ],
            out_specs=[
                pl.BlockSpec(
                    (_RB, _DC // 128 * 4, 128),
                    lambda i: (i // _GRID_C, i % _GRID_C, 0),
                )
            ],
            core_axis_name=("c", "s"),
            dimension_semantics=(pltpu.PARALLEL,),
            trace_scopes=False,
        )(x_hbm, out_hbm)

    return k(xt, tbl_pad)


@jax.jit
def kernel(x, map_table):
    tbl_strided = jnp.repeat(map_table.reshape(36), _LANES)
    p = _sc_lookup(x.T, tbl_strided)
    q = p.reshape(_S, _B // 128, 4, 128)
    return q.transpose(1, 3, 0, 2).reshape(_B, _S, 4)


# speedup vs baseline: 1.0108x; 1.0108x over previous
import dataclasses
import functools

import jax
import jax.numpy as jnp
from jax.experimental import pallas as pl
from jax.experimental.pallas import tpu as pltpu
from jax.experimental.pallas import tpu_sc as plsc

_B, _S = 16384, 200
_LANES = 16
_RB = 8
_DC = 1024
_GRID_R = _S // _RB
_GRID_C = _B // _DC
_GRID = _GRID_R * _GRID_C


def _sc_lookup(xt, tbl_pad):
    mesh = plsc.VectorSubcoreMesh(core_axis_name="c", subcore_axis_name="s")
    cp = pltpu.CompilerParams()
    if "needs_layout_passes" in pltpu.CompilerParams.__dataclass_fields__:
        cp = dataclasses.replace(cp, needs_layout_passes=False)

    @functools.partial(
        pl.kernel,
        out_type=jax.ShapeDtypeStruct((_S, _B // 32, 128), jnp.float32),
        mesh=mesh,
        scratch_types=[pltpu.VMEM((576,), jnp.float32)],
        compiler_params=cp,
    )
    def k(x_hbm, tbl_hbm, out_hbm, ts_v):
        pltpu.sync_copy(tbl_hbm, ts_v)

        def body(x_vmem, o_vmem):
            lane = jax.lax.broadcasted_iota(jnp.int32, (_LANES,), 0)
            lane_c = [lane + _LANES * c for c in range(4)]

            @plsc.parallel_loop(0, _RB, unroll=2)
            def _(r):
                @plsc.parallel_loop(0, _DC // 128, unroll=4)
                def _(h):
                    for i in range(128 // _LANES):
                        xv = x_vmem[r, pl.ds(h * 128 + i * _LANES, _LANES)]
                        base = xv * 64
                        for c in range(4):
                            vals = plsc.load_gather(ts_v, [base + lane_c[c]])
                            o_vmem[r, h * 4 + c, pl.ds(i * _LANES, _LANES)] = vals

        pltpu.emit_pipeline(
            body,
            grid=(_GRID,),
            in_specs=[
                pl.BlockSpec((_RB, _DC), lambda i: (i // _GRID_C, i % _GRID_C))
            ],
            out_specs=[
                pl.BlockSpec(
                    (_RB, _DC // 128 * 4, 128),
                    lambda i: (i // _GRID_C, i % _GRID_C, 0),
                )
            ],
            core_axis_name=("c", "s"),
            dimension_semantics=(pltpu.PARALLEL,),
            trace_scopes=False,
        )(x_hbm, out_hbm)

    return k(xt, tbl_pad)


@jax.jit
def kernel(x, map_table):
    tbl_strided = jnp.repeat(map_table.reshape(36), _LANES)
    p = _sc_lookup(x.T, tbl_strided)
    q = p.reshape(_S, _B // 128, 4, 128)
    return q.transpose(1, 3, 0, 2).reshape(_B, _S, 4)
